# Initial kernel scaffold; baseline (speedup 1.0000x reference)
import jax
import jax.numpy as jnp
from jax.experimental import pallas as pl


def kernel(features, edge_index, y_onehot, noise1, noise2, W0, b0, Wlin, blin, W1, b1, W2, b2, Wd, bd):
    raise NotImplementedError("write your pallas kernel here")


# speedup vs baseline: 3.9803x; 3.9803x over previous
import functools

import jax
import jax.numpy as jnp
from jax import lax
from jax.experimental import pallas as pl
from jax.experimental.pallas import tpu as pltpu
from jax.experimental.pallas import tpu_sc as plsc

_N = 10000
_E = 320000
_IN_DIM = 128
_H1 = 256
_H2 = 128
_NC = 16

_NCORES = 2
_NSUB = 16
_EB = 128
_NB = _E // _EB
_NPAD = 10240
_ROWS_PER_TILE = _NPAD // _NSUB
_RCHUNK = 128

_MESH = dict(core_axis_name="c", subcore_axis_name="s",
             num_cores=_NCORES, num_subcores=_NSUB)


def _zero_vmem_2d(buf, rows, cols):
    def row_body(r, carry):
        for k in range(cols // 16):
            buf[r, pl.ds(k * 16, 16)] = jnp.zeros((16,), jnp.float32)
        return carry
    lax.fori_loop(0, rows, row_body, 0)


def _zero_shared_chunk(zbuf, acc_sh, base):
    for c in range(_ROWS_PER_TILE // _RCHUNK):
        pltpu.sync_copy(zbuf, acc_sh.at[pl.ds(base + c * _RCHUNK, _RCHUNK)])


def _readout_shared_chunk(buf, acc_sh, out_hbm, base):
    for c in range(_ROWS_PER_TILE // _RCHUNK):
        pltpu.sync_copy(acc_sh.at[pl.ds(base + c * _RCHUNK, _RCHUNK)], buf)
        pltpu.sync_copy(buf, out_hbm.at[pl.ds(base + c * _RCHUNK, _RCHUNK)])


@functools.cache
def _get_sc_degrees():
    return functools.partial(
        pl.kernel,
        out_type=[jax.ShapeDtypeStruct((_NPAD, 128), jnp.float32),
                  jax.ShapeDtypeStruct((_NPAD, 128), jnp.float32)],
        mesh=plsc.VectorSubcoreMesh(**_MESH),
        scratch_types=[
            pltpu.VMEM_SHARED((_NPAD, 128), jnp.float32),
            pltpu.VMEM((_EB,), jnp.int32),
            pltpu.VMEM((_EB, 128), jnp.float32),
        ],
    )(_sc_degrees)


def _sc_degrees(src_hbm, dst_hbm, ones_hbm, osrc_hbm, odst_hbm,
                acc_sh, idx_v, ones_v):
    cid = lax.axis_index("c")
    sid = lax.axis_index("s")
    base = sid * _ROWS_PER_TILE

    _zero_vmem_2d(ones_v, _EB, 128)
    for c in range(_ROWS_PER_TILE // _RCHUNK):
        pltpu.sync_copy(ones_v, acc_sh.at[pl.ds(base + c * _RCHUNK, _RCHUNK)])
    pltpu.sync_copy(ones_hbm, ones_v)
    plsc.subcore_barrier()

    def hist(edges_hbm):
        def body(j, carry):
            b = sid + j * _NSUB

            @pl.when(b < _NB)
            def _():
                pltpu.sync_copy(edges_hbm.at[pl.ds(b * _EB, _EB)], idx_v)
                pltpu.sync_copy(ones_v, acc_sh.at[idx_v], add=True)
            return carry
        lax.fori_loop(0, (_NB + _NSUB - 1) // _NSUB, body, 0)

    pl.when(cid == 0)(lambda: hist(src_hbm))
    pl.when(cid == 1)(lambda: hist(dst_hbm))
    plsc.subcore_barrier()

    pl.when(cid == 0)(lambda: _readout_shared_chunk(ones_v, acc_sh, osrc_hbm, base))
    pl.when(cid == 1)(lambda: _readout_shared_chunk(ones_v, acc_sh, odst_hbm, base))


def _prop_body(xs_hbm, out_hbm, src_hbm, dst_hbm,
               acc_sh, sidx_v, didx_v, rows_v, buf_v, sem,
               sid, b_lo, b_hi):
    def body(j, carry):
        b = b_lo + sid + j * _NSUB

        @pl.when(b < b_hi)
        def _():
            pltpu.sync_copy(src_hbm.at[pl.ds(b * _EB, _EB)], sidx_v)
            pltpu.sync_copy(dst_hbm.at[pl.ds(b * _EB, _EB)], didx_v)
            pltpu.async_copy(xs_hbm.at[sidx_v], rows_v, sem).wait()
            pltpu.sync_copy(rows_v, acc_sh.at[didx_v], add=True)
        return carry
    nloop = (b_hi - b_lo + _NSUB - 1) // _NSUB
    lax.fori_loop(0, nloop, body, 0)


@functools.cache
def _make_prop(col_split):
    C = 128

    def prop(xs0_hbm, xs1_hbm, src_hbm, dst_hbm, o0_hbm, o1_hbm,
             acc_sh, sidx_v, didx_v, rows_v, buf_v, sem):
        cid = lax.axis_index("c")
        sid = lax.axis_index("s")
        base = sid * _ROWS_PER_TILE

        _zero_vmem_2d(buf_v, _RCHUNK, C)
        _zero_shared_chunk(buf_v, acc_sh, base)
        plsc.subcore_barrier()

        if col_split:
            pl.when(cid == 0)(lambda: _prop_body(
                xs0_hbm, o0_hbm, src_hbm, dst_hbm,
                acc_sh, sidx_v, didx_v, rows_v, buf_v, sem, sid, 0, _NB))
            pl.when(cid == 1)(lambda: _prop_body(
                xs1_hbm, o1_hbm, src_hbm, dst_hbm,
                acc_sh, sidx_v, didx_v, rows_v, buf_v, sem, sid, 0, _NB))
        else:
            pl.when(cid == 0)(lambda: _prop_body(
                xs0_hbm, o0_hbm, src_hbm, dst_hbm,
                acc_sh, sidx_v, didx_v, rows_v, buf_v, sem, sid, 0, _NB // 2))
            pl.when(cid == 1)(lambda: _prop_body(
                xs1_hbm, o1_hbm, src_hbm, dst_hbm,
                acc_sh, sidx_v, didx_v, rows_v, buf_v, sem, sid, _NB // 2, _NB))

        plsc.subcore_barrier()
        pl.when(cid == 0)(lambda: _readout_shared_chunk(buf_v, acc_sh, o0_hbm, base))
        pl.when(cid == 1)(lambda: _readout_shared_chunk(buf_v, acc_sh, o1_hbm, base))

    return functools.partial(
        pl.kernel,
        out_type=[jax.ShapeDtypeStruct((_NPAD, C), jnp.float32),
                  jax.ShapeDtypeStruct((_NPAD, C), jnp.float32)],
        mesh=plsc.VectorSubcoreMesh(**_MESH),
        scratch_types=[
            pltpu.VMEM_SHARED((_NPAD, C), jnp.float32),
            pltpu.VMEM((_EB,), jnp.int32),
            pltpu.VMEM((_EB,), jnp.int32),
            pltpu.VMEM((_EB, C), jnp.float32),
            pltpu.VMEM((_RCHUNK, C), jnp.float32),
            pltpu.SemaphoreType.DMA,
        ],
    )(prop)


def _row_scale(hist_blk):
    return lax.rsqrt(jnp.maximum(hist_blk[:, 0:1], 1.0))


def _tc_scale_kernel(f_ref, hs_ref, o_ref):
    o_ref[...] = f_ref[...] * _row_scale(hs_ref[...])


def _tc_encode_kernel(p0a_ref, p0b_ref, hd_ref, hs_ref, y_ref,
                      w0_ref, b0_ref, wlin_ref, blin_ref, wcat_ref,
                      h_ref, glo_ref, ghi_ref):
    s_in = _row_scale(hd_ref[...])
    s_out = _row_scale(hs_ref[...])
    p0 = (p0a_ref[...] + p0b_ref[...]) * s_in
    h0 = jnp.maximum(jnp.dot(p0, w0_ref[...],
                             preferred_element_type=jnp.float32) + b0_ref[...], 0.0)
    hcat = jnp.concatenate([h0, y_ref[...]], axis=1)
    h = jnp.maximum(jnp.dot(hcat, wlin_ref[...],
                            preferred_element_type=jnp.float32) + blin_ref[...], 0.0)
    g = jnp.dot(h, wcat_ref[...], preferred_element_type=jnp.float32) * s_out
    h_ref[...] = h
    glo_ref[...] = g[:, :128]
    ghi_ref[...] = g[:, 128:]


def _l2norm(x):
    n = jnp.sqrt(jnp.sum(x * x, axis=-1, keepdims=True))
    return x / jnp.maximum(n, 1e-12)


def _tc_decode_kernel(qlo_ref, qhi_ref, hd_ref, b1_ref, b2_ref,
                      n1_ref, n2_ref, y_ref, wd_ref, bd_ref,
                      mean_ref, ls_ref, z1_ref, z2_ref,
                      f1_ref, f2_ref, kl_ref, zc1_ref):
    s_in = _row_scale(hd_ref[...])
    mean = _l2norm(qlo_ref[...] * s_in + b1_ref[...])
    log_std = _l2norm(qhi_ref[...] * s_in + b2_ref[...])
    e = jnp.exp(log_std)
    z1 = mean + n1_ref[...] * e
    z2 = mean + n2_ref[...] * e
    y = y_ref[...]
    zc1 = jnp.concatenate([z1, y], axis=1)
    zc2 = jnp.concatenate([z2, y], axis=1)
    f1 = jnp.maximum(jnp.dot(zc1, wd_ref[...],
                             preferred_element_type=jnp.float32) + bd_ref[...], 0.0)
    f2 = jnp.maximum(jnp.dot(zc2, wd_ref[...],
                             preferred_element_type=jnp.float32) + bd_ref[...], 0.0)
    d = z1 - mean
    kl = jnp.sum(0.5 * (z1 * z1 - log_std - d * d / e), axis=-1, keepdims=True)
    mean_ref[...] = mean
    ls_ref[...] = log_std
    z1_ref[...] = z1
    z2_ref[...] = z2
    f1_ref[...] = f1
    f2_ref[...] = f2
    kl_ref[...] = kl
    zc1_ref[...] = zc1


def _tc_adj_kernel(a_ref, bt_ref, o_ref):
    o_ref[...] = jax.nn.sigmoid(
        jnp.dot(a_ref[...], bt_ref[...], preferred_element_type=jnp.float32))


def _full(shape):
    return pl.BlockSpec(shape, lambda *_: (0,) * len(shape))


def _rows(tm, cols):
    return pl.BlockSpec((tm, cols), lambda i: (i, 0))


def kernel(features, edge_index, y_onehot, noise1, noise2,
           W0, b0, Wlin, blin, W1, b1, W2, b2, Wd, bd):
    src = edge_index[0]
    dst = edge_index[1]

    hist_src, hist_dst = _get_sc_degrees()(
        src, dst, jnp.ones((_EB, 128), jnp.float32))

    TM = 1000
    grid = (_N // TM,)
    xs0 = pl.pallas_call(
        _tc_scale_kernel,
        grid=grid,
        in_specs=[_rows(TM, _IN_DIM), _rows(TM, 128)],
        out_specs=_rows(TM, _IN_DIM),
        out_shape=jax.ShapeDtypeStruct((_N, _IN_DIM), jnp.float32),
    )(features, hist_src)

    p0a, p0b = _make_prop(col_split=False)(xs0, xs0, src, dst)

    Wcat = jnp.concatenate([W1, W2], axis=1)
    h, glo, ghi = pl.pallas_call(
        _tc_encode_kernel,
        grid=grid,
        in_specs=[_rows(TM, _IN_DIM), _rows(TM, _IN_DIM),
                  _rows(TM, 128), _rows(TM, 128), _rows(TM, _NC),
                  _full((_IN_DIM, _H1)), _full((1, _H1)),
                  _full((_H1 + _NC, _H1 + _NC)), _full((1, _H1 + _NC)),
                  _full((_H1 + _NC, 2 * _H2))],
        out_specs=[_rows(TM, _H1 + _NC), _rows(TM, _H2), _rows(TM, _H2)],
        out_shape=[jax.ShapeDtypeStruct((_N, _H1 + _NC), jnp.float32),
                   jax.ShapeDtypeStruct((_N, _H2), jnp.float32),
                   jax.ShapeDtypeStruct((_N, _H2), jnp.float32)],
    )(p0a, p0b, hist_dst, hist_src, y_onehot,
      W0, b0.reshape(1, -1), Wlin, blin.reshape(1, -1), Wcat)

    qlo, qhi = _make_prop(col_split=True)(glo, ghi, src, dst)

    (mean, log_std, z1, z2, fea_rec1, fea_rec2, kl2d, zc1) = pl.pallas_call(
        _tc_decode_kernel,
        grid=grid,
        in_specs=[_rows(TM, _H2), _rows(TM, _H2), _rows(TM, 128),
                  _full((1, _H2)), _full((1, _H2)),
                  _rows(TM, _H2), _rows(TM, _H2), _rows(TM, _NC),
                  _full((_H2 + _NC, _H1)), _full((1, _H1))],
        out_specs=[_rows(TM, _H2), _rows(TM, _H2), _rows(TM, _H2),
                   _rows(TM, _H2), _rows(TM, _H1), _rows(TM, _H1),
                   _rows(TM, 1), _rows(TM, _H2 + _NC)],
        out_shape=[jax.ShapeDtypeStruct((_N, _H2), jnp.float32),
                   jax.ShapeDtypeStruct((_N, _H2), jnp.float32),
                   jax.ShapeDtypeStruct((_N, _H2), jnp.float32),
                   jax.ShapeDtypeStruct((_N, _H2), jnp.float32),
                   jax.ShapeDtypeStruct((_N, _H1), jnp.float32),
                   jax.ShapeDtypeStruct((_N, _H1), jnp.float32),
                   jax.ShapeDtypeStruct((_N, 1), jnp.float32),
                   jax.ShapeDtypeStruct((_N, _H2 + _NC), jnp.float32)],
    )(qlo, qhi, hist_dst, b1.reshape(1, -1), b2.reshape(1, -1),
      noise1, noise2, y_onehot, Wd, bd.reshape(1, -1))

    zc1T = zc1.T
    AM, AN = 512, 1024
    adj_rec = pl.pallas_call(
        _tc_adj_kernel,
        grid=(pl.cdiv(_N, AM), pl.cdiv(_N, AN)),
        in_specs=[pl.BlockSpec((AM, _H2 + _NC), lambda i, j: (i, 0)),
                  pl.BlockSpec((_H2 + _NC, AN), lambda i, j: (0, j))],
        out_specs=pl.BlockSpec((AM, AN), lambda i, j: (i, j)),
        out_shape=jax.ShapeDtypeStruct((_N, _N), jnp.float32),
    )(zc1, zc1T)

    return (adj_rec, z1, z2, h, fea_rec1, fea_rec2, mean, log_std,
            kl2d.reshape(_N))
